# baseline (speedup 1.0000x reference)
import functools

import jax
import jax.numpy as jnp
from jax import lax
from jax.experimental import pallas as pl
from jax.experimental.pallas import tpu as pltpu
from jax.experimental.pallas import tpu_sc as plsc

BATCH = 16384
EMBED_DIM = 32
NUM_CORES = 2
NUM_SUBCORES = 16
LANES = 16
NUM_WORKERS = NUM_CORES * NUM_SUBCORES
TC_ROWS = 8192
SC_ROWS = BATCH - TC_ROWS
B_PER_W = SC_ROWS // NUM_WORKERS
NUM_GROUPS = B_PER_W // LANES


def _dot_kernel(uid_hbm, iid_hbm, ut_hbm, it_hbm, out_hbm,
                uidx_v, iidx_v, urows_v, irows_v, out_v, sem_u, sem_i):
    wid = lax.axis_index("s") * NUM_CORES + lax.axis_index("c")
    base = pl.multiple_of(wid * B_PER_W, B_PER_W)

    pltpu.sync_copy(uid_hbm.at[pl.ds(base, B_PER_W)], uidx_v)
    pltpu.sync_copy(iid_hbm.at[pl.ds(base, B_PER_W)], iidx_v)

    lane_iota = lax.iota(jnp.int32, LANES)

    @plsc.parallel_loop(0, NUM_GROUPS, unroll=4)
    def _issue(g):
        gstart = pl.multiple_of(g * LANES, LANES)
        u16 = uidx_v[pl.ds(gstart, LANES)]
        i16 = iidx_v[pl.ds(gstart, LANES)]
        for l in range(LANES):
            j = g * LANES + l
            pltpu.async_copy(ut_hbm.at[u16[l]], urows_v.at[j], sem_u)
            pltpu.async_copy(it_hbm.at[i16[l]], irows_v.at[j], sem_i)

    pltpu.make_async_copy(ut_hbm.at[pl.ds(0, B_PER_W)], urows_v,
                          sem_u).wait()
    pltpu.make_async_copy(ut_hbm.at[pl.ds(0, B_PER_W)], irows_v,
                          sem_i).wait()

    def block_body(blk, _):
        row_idx = blk * LANES + lane_iota
        acc = jnp.zeros((LANES,), jnp.float32)
        for d in range(EMBED_DIM):
            col_idx = jnp.full((LANES,), d, jnp.int32)
            u = plsc.load_gather(urows_v, [row_idx, col_idx])
            v = plsc.load_gather(irows_v, [row_idx, col_idx])
            acc = acc + u * v
        start = pl.multiple_of(blk * LANES, LANES)
        out_v[pl.ds(start, LANES)] = acc
        return _

    lax.fori_loop(0, NUM_GROUPS, block_body, None)

    pltpu.sync_copy(out_v, out_hbm.at[pl.ds(base, B_PER_W)])


def _sc_part(user_ids, item_ids, user_table, item_table):
    mesh = plsc.VectorSubcoreMesh(core_axis_name="c", subcore_axis_name="s")
    return pl.kernel(
        _dot_kernel,
        mesh=mesh,
        out_type=jax.ShapeDtypeStruct((SC_ROWS,), jnp.float32),
        scratch_types=[
            pltpu.VMEM((B_PER_W,), jnp.int32),
            pltpu.VMEM((B_PER_W,), jnp.int32),
            pltpu.VMEM((B_PER_W, EMBED_DIM), jnp.float32),
            pltpu.VMEM((B_PER_W, EMBED_DIM), jnp.float32),
            pltpu.VMEM((B_PER_W,), jnp.float32),
            pltpu.SemaphoreType.DMA,
            pltpu.SemaphoreType.DMA,
        ],
        compiler_params=pltpu.CompilerParams(needs_layout_passes=False),
    )(user_ids, item_ids, user_table, item_table)


def _tc_kernel(uid_smem, iid_smem, ut_any, it_any, o_ref,
               urows_v, irows_v, sem_u, sem_i):
    def issue(j, _):
        pltpu.make_async_copy(ut_any.at[uid_smem[j]], urows_v.at[j],
                              sem_u).start()
        pltpu.make_async_copy(it_any.at[iid_smem[j]], irows_v.at[j],
                              sem_i).start()
        return _

    lax.fori_loop(0, TC_ROWS, issue, None, unroll=8)

    pltpu.make_async_copy(ut_any.at[pl.ds(0, TC_ROWS)], urows_v,
                          sem_u).wait()
    pltpu.make_async_copy(ut_any.at[pl.ds(0, TC_ROWS)], irows_v,
                          sem_i).wait()

    o_ref[...] = jnp.sum(urows_v[...] * irows_v[...], axis=1)


def _tc_part(user_ids, item_ids, user_table, item_table):
    return pl.pallas_call(
        _tc_kernel,
        in_specs=[
            pl.BlockSpec(memory_space=pltpu.SMEM),
            pl.BlockSpec(memory_space=pltpu.SMEM),
            pl.BlockSpec(memory_space=pltpu.MemorySpace.HBM),
            pl.BlockSpec(memory_space=pltpu.MemorySpace.HBM),
        ],
        out_specs=pl.BlockSpec(memory_space=pltpu.VMEM),
        out_shape=jax.ShapeDtypeStruct((TC_ROWS,), jnp.float32),
        scratch_shapes=[
            pltpu.VMEM((TC_ROWS, EMBED_DIM), jnp.float32),
            pltpu.VMEM((TC_ROWS, EMBED_DIM), jnp.float32),
            pltpu.SemaphoreType.DMA,
            pltpu.SemaphoreType.DMA,
        ],
    )(user_ids, item_ids, user_table, item_table)


@jax.jit
def _run(user_ids, item_ids, user_table, item_table):
    sc_out = _sc_part(user_ids[:SC_ROWS], item_ids[:SC_ROWS],
                      user_table, item_table)
    tc_out = _tc_part(user_ids[SC_ROWS:], item_ids[SC_ROWS:],
                      user_table, item_table)
    return jnp.concatenate([sc_out, tc_out])


def kernel(user_ids, item_ids, user_table, item_table):
    return _run(user_ids.astype(jnp.int32), item_ids.astype(jnp.int32),
                user_table, item_table)


# speedup vs baseline: 1.0835x; 1.0835x over previous
import functools

import jax
import jax.numpy as jnp
from jax import lax
from jax.experimental import pallas as pl
from jax.experimental.pallas import tpu as pltpu
from jax.experimental.pallas import tpu_sc as plsc

BATCH = 16384
EMBED_DIM = 32
NUM_CORES = 2
NUM_SUBCORES = 16
LANES = 16
NUM_WORKERS = NUM_CORES * NUM_SUBCORES
B_PER_W = BATCH // NUM_WORKERS
CHUNK = 256
GROUPS_PER_CHUNK = CHUNK // LANES
NSEM = 1
ROWS_PER_SEM = CHUNK // NSEM


def _dot_kernel(uid_hbm, iid_hbm, ut_hbm, it_hbm, out_hbm,
                uidx_v, iidx_v, urows_v, irows_v, out_v, *sems):
    usems = sems[:NSEM]
    isems = sems[NSEM:]
    wid = lax.axis_index("s") * NUM_CORES + lax.axis_index("c")
    base = pl.multiple_of(wid * B_PER_W, B_PER_W)

    pltpu.sync_copy(uid_hbm.at[pl.ds(base, B_PER_W)], uidx_v)
    pltpu.sync_copy(iid_hbm.at[pl.ds(base, B_PER_W)], iidx_v)

    lane_iota = lax.iota(jnp.int32, LANES)

    def chunk_body(ck, _):
        cbase = pl.multiple_of(ck * CHUNK, CHUNK)

        @plsc.parallel_loop(0, GROUPS_PER_CHUNK, unroll=4)
        def _issue(g):
            gstart = pl.multiple_of(cbase + g * LANES, LANES)
            u16 = uidx_v[pl.ds(gstart, LANES)]
            i16 = iidx_v[pl.ds(gstart, LANES)]
            for l in range(LANES):
                j = g * LANES + l
                pltpu.async_copy(ut_hbm.at[u16[l]], urows_v.at[j],
                                 usems[l % NSEM])
                pltpu.async_copy(it_hbm.at[i16[l]], irows_v.at[j],
                                 isems[l % NSEM])

        for k in range(NSEM):
            pltpu.make_async_copy(ut_hbm.at[pl.ds(0, ROWS_PER_SEM)],
                                  urows_v.at[pl.ds(0, ROWS_PER_SEM)],
                                  usems[k]).wait()
            pltpu.make_async_copy(ut_hbm.at[pl.ds(0, ROWS_PER_SEM)],
                                  irows_v.at[pl.ds(0, ROWS_PER_SEM)],
                                  isems[k]).wait()

        def block_body(blk, _):
            row_idx = blk * LANES + lane_iota
            acc = jnp.zeros((LANES,), jnp.float32)
            for d in range(EMBED_DIM):
                col_idx = jnp.full((LANES,), d, jnp.int32)
                u = plsc.load_gather(urows_v, [row_idx, col_idx])
                v = plsc.load_gather(irows_v, [row_idx, col_idx])
                acc = acc + u * v
            start = pl.multiple_of(cbase + blk * LANES, LANES)
            out_v[pl.ds(start, LANES)] = acc
            return _

        lax.fori_loop(0, GROUPS_PER_CHUNK, block_body, None)
        return _

    lax.fori_loop(0, B_PER_W // CHUNK, chunk_body, None)

    pltpu.sync_copy(out_v, out_hbm.at[pl.ds(base, B_PER_W)])


@jax.jit
def _run(user_ids, item_ids, user_table, item_table):
    mesh = plsc.VectorSubcoreMesh(core_axis_name="c", subcore_axis_name="s")
    return pl.kernel(
        _dot_kernel,
        mesh=mesh,
        out_type=jax.ShapeDtypeStruct((BATCH,), jnp.float32),
        scratch_types=[
            pltpu.VMEM((B_PER_W,), jnp.int32),
            pltpu.VMEM((B_PER_W,), jnp.int32),
            pltpu.VMEM((CHUNK, EMBED_DIM), jnp.float32),
            pltpu.VMEM((CHUNK, EMBED_DIM), jnp.float32),
            pltpu.VMEM((B_PER_W,), jnp.float32),
        ] + [pltpu.SemaphoreType.DMA] * (2 * NSEM),
        compiler_params=pltpu.CompilerParams(needs_layout_passes=False),
    )(user_ids, item_ids, user_table, item_table)


def kernel(user_ids, item_ids, user_table, item_table):
    return _run(user_ids.astype(jnp.int32), item_ids.astype(jnp.int32),
                user_table, item_table)
